# baseline (speedup 1.0000x reference)
import functools

import jax
import jax.numpy as jnp
from jax import lax
from jax.experimental import pallas as pl
from jax.experimental.pallas import tpu as pltpu
from jax.experimental.pallas import tpu_sc as plsc

VOCAB = 100000
HIDDEN = 768
EPS = 1e-05
LANES = 16
NBLK = HIDDEN // LANES

NC = 2
NS = 16
NW = NC * NS

B_TOTAL = 4 * 8192
BPW = B_TOTAL // NW
CHUNK = 32
NCHUNK = BPW // CHUNK
NBUF = 4
NSTEP = NCHUNK // NBUF
GROUP = 32
NGROUP = CHUNK // GROUP


def _rsqrt(x):
    i = lax.bitcast_convert_type(x, jnp.int32)
    i = jnp.int32(0x5F3759DF) - (i >> 1)
    y = lax.bitcast_convert_type(i, jnp.float32)
    for _ in range(2):
        y = y * (jnp.float32(1.5) - jnp.float32(0.5) * x * y * y)
    return y


def _sc_embed_ln(ids_hbm, table_hbm, gamma_hbm, beta_hbm, out_hbm,
                 idx_v, rows0, rows1, rows2, rows3,
                 gamma_v, beta_v, mean_s, rstd_s,
                 si0, si1, si2, si3, so0, so1, so2, so3):
    rows = (rows0, rows1, rows2, rows3)
    sem_in = (si0, si1, si2, si3)
    sem_out = (so0, so1, so2, so3)

    wid = lax.axis_index("s") * NC + lax.axis_index("c")
    base = wid * BPW
    pltpu.sync_copy(gamma_hbm, gamma_v)
    pltpu.sync_copy(beta_hbm, beta_v)
    pltpu.sync_copy(ids_hbm.at[pl.ds(base, BPW)], idx_v)

    inv_h = jnp.float32(1.0 / HIDDEN)

    def gather_start(ch, b):
        pltpu.async_copy(
            table_hbm.at[idx_v.at[pl.ds(ch * CHUNK, CHUNK)]], rows[b], sem_in[b]
        )

    def gather_wait(ch, b):
        pltpu.make_async_copy(
            table_hbm.at[idx_v.at[pl.ds(ch * CHUNK, CHUNK)]], rows[b], sem_in[b]
        ).wait()

    def out_start(ch, b):
        pltpu.async_copy(
            rows[b], out_hbm.at[pl.ds(base + ch * CHUNK, CHUNK)], sem_out[b]
        )

    def out_wait(ch, b):
        pltpu.make_async_copy(
            rows[b], out_hbm.at[pl.ds(base + ch * CHUNK, CHUNK)], sem_out[b]
        ).wait()

    def compute(b):
        rv = rows[b]

        def stats_body(r, _):
            zero = jnp.zeros((LANES,), jnp.float32)
            s0, s1, q0, q1 = zero, zero, zero, zero
            for j in range(NBLK):
                x = rv[r, pl.ds(j * LANES, LANES)]
                if j % 2 == 0:
                    s0 = s0 + x
                    q0 = q0 + x * x
                else:
                    s1 = s1 + x
                    q1 = q1 + x * x
            ssum = plsc.cumsum(s0 + s1)[LANES - 1]
            qsum = plsc.cumsum(q0 + q1)[LANES - 1]
            mean = ssum * inv_h
            var = qsum * inv_h - mean * mean
            a = _rsqrt(var + jnp.float32(EPS))
            rstd_s[r] = a
            mean_s[r] = -mean * a
            return 0

        lax.fori_loop(0, CHUNK, stats_body, 0)

        for grp in range(NGROUP):
            rb = grp * GROUP
            na = [mean_s[rb + r] for r in range(GROUP)]
            a = [rstd_s[rb + r] for r in range(GROUP)]

            def norm_body(j, _):
                sl = pl.ds(j * LANES, LANES)
                g = gamma_v[sl]
                bb = beta_v[sl]
                for r in range(GROUP):
                    x = rv[rb + r, sl]
                    rv[rb + r, sl] = (x * a[r] + na[r]) * g + bb
                return 0

            lax.fori_loop(0, NBLK, norm_body, 0)

    for b in range(NBUF - 1):
        gather_start(b, b)

    def step_body(ci, _):
        for b in range(NBUF):
            ch = ci * NBUF + b
            wb = (b + NBUF - 1) % NBUF
            if b == 0:
                @pl.when(ci > 0)
                def _():
                    out_wait(ch - 1, wb)
                gather_start(ch + NBUF - 1, wb)
            else:
                out_wait(ch - 1, wb)

                @pl.when(ci < NSTEP - 1)
                def _():
                    gather_start(ch + NBUF - 1, wb)
            gather_wait(ch, b)
            compute(b)
            out_start(ch, b)
        return 0

    lax.fori_loop(0, NSTEP, step_body, 0)
    out_wait(NCHUNK - 1, NBUF - 1)


@jax.jit
def kernel(input_ids, word_embeddings, ln_gamma, ln_beta):
    ids_flat = input_ids.reshape(-1).astype(jnp.int32)
    mesh = plsc.VectorSubcoreMesh(
        core_axis_name="c", subcore_axis_name="s", num_cores=NC, num_subcores=NS
    )
    k = pl.kernel(
        _sc_embed_ln,
        out_type=jax.ShapeDtypeStruct((B_TOTAL, HIDDEN), jnp.float32),
        mesh=mesh,
        compiler_params=pltpu.CompilerParams(needs_layout_passes=False),
        scratch_types=[
            pltpu.VMEM((BPW,), jnp.int32),
            pltpu.VMEM((CHUNK, HIDDEN), jnp.float32),
            pltpu.VMEM((CHUNK, HIDDEN), jnp.float32),
            pltpu.VMEM((CHUNK, HIDDEN), jnp.float32),
            pltpu.VMEM((CHUNK, HIDDEN), jnp.float32),
            pltpu.VMEM((HIDDEN,), jnp.float32),
            pltpu.VMEM((HIDDEN,), jnp.float32),
            pltpu.SMEM((CHUNK,), jnp.float32),
            pltpu.SMEM((CHUNK,), jnp.float32),
            pltpu.SemaphoreType.DMA,
            pltpu.SemaphoreType.DMA,
            pltpu.SemaphoreType.DMA,
            pltpu.SemaphoreType.DMA,
            pltpu.SemaphoreType.DMA,
            pltpu.SemaphoreType.DMA,
            pltpu.SemaphoreType.DMA,
            pltpu.SemaphoreType.DMA,
        ],
    )
    out = k(ids_flat, word_embeddings, ln_gamma, ln_beta)
    return out.reshape(input_ids.shape + (HIDDEN,))


# speedup vs baseline: 1.0597x; 1.0597x over previous
import functools

import jax
import jax.numpy as jnp
from jax import lax
from jax.experimental import pallas as pl
from jax.experimental.pallas import tpu as pltpu
from jax.experimental.pallas import tpu_sc as plsc

VOCAB = 100000
HIDDEN = 768
EPS = 1e-05
LANES = 16
NBLK = HIDDEN // LANES

NC = 2
NS = 16
NW = NC * NS

B_TOTAL = 4 * 8192
BPW = B_TOTAL // NW
CHUNK = 32
NCHUNK = BPW // CHUNK
NBUF = 4
NSTEP = NCHUNK // NBUF
GROUP = 16
NGROUP = CHUNK // GROUP


def _rsqrt(x):
    i = lax.bitcast_convert_type(x, jnp.int32)
    i = jnp.int32(0x5F3759DF) - (i >> 1)
    y = lax.bitcast_convert_type(i, jnp.float32)
    for _ in range(2):
        y = y * (jnp.float32(1.5) - jnp.float32(0.5) * x * y * y)
    return y


def _sc_embed_ln(ids_hbm, table_hbm, gamma_hbm, beta_hbm, out_hbm,
                 idx_v, rows0, rows1, rows2, rows3,
                 gamma_v, beta_v, mean_s, rstd_s,
                 si0, si1, si2, si3, so0, so1, so2, so3):
    rows = (rows0, rows1, rows2, rows3)
    sem_in = (si0, si1, si2, si3)
    sem_out = (so0, so1, so2, so3)

    wid = lax.axis_index("s") * NC + lax.axis_index("c")
    base = wid * BPW
    pltpu.sync_copy(gamma_hbm, gamma_v)
    pltpu.sync_copy(beta_hbm, beta_v)
    pltpu.sync_copy(ids_hbm.at[pl.ds(base, BPW)], idx_v)

    inv_h = jnp.float32(1.0 / HIDDEN)

    def gather_start(ch, b):
        pltpu.async_copy(
            table_hbm.at[idx_v.at[pl.ds(ch * CHUNK, CHUNK)]], rows[b], sem_in[b]
        )

    def gather_wait(ch, b):
        pltpu.make_async_copy(
            table_hbm.at[idx_v.at[pl.ds(ch * CHUNK, CHUNK)]], rows[b], sem_in[b]
        ).wait()

    def out_start(ch, b):
        pltpu.async_copy(
            rows[b], out_hbm.at[pl.ds(base + ch * CHUNK, CHUNK)], sem_out[b]
        )

    def out_wait(ch, b):
        pltpu.make_async_copy(
            rows[b], out_hbm.at[pl.ds(base + ch * CHUNK, CHUNK)], sem_out[b]
        ).wait()

    def compute(b):
        rv = rows[b]

        def stats_body(r, _):
            zero = jnp.zeros((LANES,), jnp.float32)
            s0, s1, q0, q1 = zero, zero, zero, zero
            for j in range(NBLK):
                x = rv[r, pl.ds(j * LANES, LANES)]
                if j % 2 == 0:
                    s0 = s0 + x
                    q0 = q0 + x * x
                else:
                    s1 = s1 + x
                    q1 = q1 + x * x
            ssum = plsc.cumsum(s0 + s1)[LANES - 1]
            qsum = plsc.cumsum(q0 + q1)[LANES - 1]
            mean = ssum * inv_h
            var = qsum * inv_h - mean * mean
            a = _rsqrt(var + jnp.float32(EPS))
            rstd_s[r] = a
            mean_s[r] = -mean * a
            return 0

        lax.fori_loop(0, CHUNK, stats_body, 0)

        for grp in range(NGROUP):
            rb = grp * GROUP
            na = [mean_s[rb + r] for r in range(GROUP)]
            a = [rstd_s[rb + r] for r in range(GROUP)]

            def norm_body(j, _):
                sl = pl.ds(j * LANES, LANES)
                g = gamma_v[sl]
                bb = beta_v[sl]
                for r in range(GROUP):
                    x = rv[rb + r, sl]
                    rv[rb + r, sl] = (x * a[r] + na[r]) * g + bb
                return 0

            lax.fori_loop(0, NBLK, norm_body, 0)

    for b in range(NBUF - 1):
        gather_start(b, b)

    def step_body(ci, _):
        for b in range(NBUF):
            ch = ci * NBUF + b
            wb = (b + NBUF - 1) % NBUF
            if b == 0:
                @pl.when(ci > 0)
                def _():
                    out_wait(ch - 1, wb)
                gather_start(ch + NBUF - 1, wb)
            else:
                out_wait(ch - 1, wb)

                @pl.when(ci < NSTEP - 1)
                def _():
                    gather_start(ch + NBUF - 1, wb)
            gather_wait(ch, b)
            compute(b)
            out_start(ch, b)
        return 0

    lax.fori_loop(0, NSTEP, step_body, 0)
    out_wait(NCHUNK - 1, NBUF - 1)


@jax.jit
def kernel(input_ids, word_embeddings, ln_gamma, ln_beta):
    ids_flat = input_ids.reshape(-1).astype(jnp.int32)
    mesh = plsc.VectorSubcoreMesh(
        core_axis_name="c", subcore_axis_name="s", num_cores=NC, num_subcores=NS
    )
    k = pl.kernel(
        _sc_embed_ln,
        out_type=jax.ShapeDtypeStruct((B_TOTAL, HIDDEN), jnp.float32),
        mesh=mesh,
        compiler_params=pltpu.CompilerParams(needs_layout_passes=False),
        scratch_types=[
            pltpu.VMEM((BPW,), jnp.int32),
            pltpu.VMEM((CHUNK, HIDDEN), jnp.float32),
            pltpu.VMEM((CHUNK, HIDDEN), jnp.float32),
            pltpu.VMEM((CHUNK, HIDDEN), jnp.float32),
            pltpu.VMEM((CHUNK, HIDDEN), jnp.float32),
            pltpu.VMEM((HIDDEN,), jnp.float32),
            pltpu.VMEM((HIDDEN,), jnp.float32),
            pltpu.SMEM((CHUNK,), jnp.float32),
            pltpu.SMEM((CHUNK,), jnp.float32),
            pltpu.SemaphoreType.DMA,
            pltpu.SemaphoreType.DMA,
            pltpu.SemaphoreType.DMA,
            pltpu.SemaphoreType.DMA,
            pltpu.SemaphoreType.DMA,
            pltpu.SemaphoreType.DMA,
            pltpu.SemaphoreType.DMA,
            pltpu.SemaphoreType.DMA,
        ],
    )
    out = k(ids_flat, word_embeddings, ln_gamma, ln_beta)
    return out.reshape(input_ids.shape + (HIDDEN,))


# speedup vs baseline: 1.0653x; 1.0053x over previous
import jax
import jax.numpy as jnp
from jax import lax
from jax.experimental import pallas as pl
from jax.experimental.pallas import tpu as pltpu
from jax.experimental.pallas import tpu_sc as plsc

VOCAB = 100000
HIDDEN = 768
EPS = 1e-05
LANES = 16
NBLK = HIDDEN // LANES

NC = 2
NS = 16
NW = NC * NS

B_TOTAL = 4 * 8192
BPW = B_TOTAL // NW
CHUNK = 32
NCHUNK = BPW // CHUNK
NBUF = 4
NSTEP = NCHUNK // NBUF
GROUP = 16
NGROUP = CHUNK // GROUP


def _rsqrt(x):
    i = lax.bitcast_convert_type(x, jnp.int32)
    i = jnp.int32(0x5F3759DF) - (i >> 1)
    y = lax.bitcast_convert_type(i, jnp.float32)
    for _ in range(2):
        y = y * (jnp.float32(1.5) - jnp.float32(0.5) * x * y * y)
    return y


def _sc_embed_ln(ids_hbm, table_hbm, gamma_hbm, beta_hbm, out_hbm,
                 idx_v, rows0, rows1, rows2, rows3,
                 gamma_v, beta_v, mean_s, rstd_s,
                 si0, si1, si2, si3, so0, so1, so2, so3):
    rows = (rows0, rows1, rows2, rows3)
    sem_in = (si0, si1, si2, si3)
    sem_out = (so0, so1, so2, so3)

    wid = lax.axis_index("s") * NC + lax.axis_index("c")
    base = wid * BPW
    pltpu.sync_copy(gamma_hbm, gamma_v)
    pltpu.sync_copy(beta_hbm, beta_v)
    pltpu.sync_copy(ids_hbm.at[pl.ds(base, BPW)], idx_v)

    inv_h = jnp.float32(1.0 / HIDDEN)

    def gather_start(ch, b):
        pltpu.async_copy(
            table_hbm.at[idx_v.at[pl.ds(ch * CHUNK, CHUNK)]], rows[b], sem_in[b]
        )

    def gather_wait(ch, b):
        pltpu.make_async_copy(
            table_hbm.at[idx_v.at[pl.ds(ch * CHUNK, CHUNK)]], rows[b], sem_in[b]
        ).wait()

    def out_start(ch, b):
        pltpu.async_copy(
            rows[b], out_hbm.at[pl.ds(base + ch * CHUNK, CHUNK)], sem_out[b]
        )

    def out_wait(ch, b):
        pltpu.make_async_copy(
            rows[b], out_hbm.at[pl.ds(base + ch * CHUNK, CHUNK)], sem_out[b]
        ).wait()

    def compute(b):
        rv = rows[b]

        def stats_body(r, _):
            zero = jnp.zeros((LANES,), jnp.float32)
            s0, s1, q0, q1 = zero, zero, zero, zero
            for j in range(NBLK):
                x = rv[r, pl.ds(j * LANES, LANES)]
                if j % 2 == 0:
                    s0 = s0 + x
                    q0 = q0 + x * x
                else:
                    s1 = s1 + x
                    q1 = q1 + x * x
            ssum = plsc.cumsum(s0 + s1)[LANES - 1]
            qsum = plsc.cumsum(q0 + q1)[LANES - 1]
            mean = ssum * inv_h
            var = qsum * inv_h - mean * mean
            a = _rsqrt(var + jnp.float32(EPS))
            rstd_s[r] = a
            mean_s[r] = -mean * a
            return 0

        lax.fori_loop(0, CHUNK, stats_body, 0)

        for grp in range(NGROUP):
            rb = grp * GROUP
            na = [mean_s[rb + r] for r in range(GROUP)]
            a = [rstd_s[rb + r] for r in range(GROUP)]

            def norm_body(j, _):
                sl = pl.ds(j * LANES, LANES)
                g = gamma_v[sl]
                bb = beta_v[sl]
                for r in range(GROUP):
                    x = rv[rb + r, sl]
                    rv[rb + r, sl] = (x * a[r] + na[r]) * g + bb
                return 0

            lax.fori_loop(0, NBLK, norm_body, 0)

    for b in range(NBUF - 1):
        gather_start(b, b)

    def step_body(ci, _):
        for b in range(NBUF):
            ch = ci * NBUF + b
            wb = (b + NBUF - 1) % NBUF
            if b == 0:
                @pl.when(ci > 0)
                def _():
                    out_wait(ch - 1, wb)
                gather_start(ch + NBUF - 1, wb)
            else:
                out_wait(ch - 1, wb)

                @pl.when(ci < NSTEP - 1)
                def _():
                    gather_start(ch + NBUF - 1, wb)
            gather_wait(ch, b)
            compute(b)
            out_start(ch, b)
        return 0

    lax.fori_loop(0, NSTEP, step_body, 0)
    out_wait(NCHUNK - 1, NBUF - 1)


@jax.jit
def kernel(input_ids, word_embeddings, ln_gamma, ln_beta):
    ids_flat = input_ids.reshape(-1).astype(jnp.int32)
    mesh = plsc.VectorSubcoreMesh(
        core_axis_name="c", subcore_axis_name="s", num_cores=NC, num_subcores=NS
    )
    k = pl.kernel(
        _sc_embed_ln,
        out_type=jax.ShapeDtypeStruct((B_TOTAL, HIDDEN), jnp.float32),
        mesh=mesh,
        compiler_params=pltpu.CompilerParams(needs_layout_passes=False),
        scratch_types=[
            pltpu.VMEM((BPW,), jnp.int32),
            pltpu.VMEM((CHUNK, HIDDEN), jnp.float32),
            pltpu.VMEM((CHUNK, HIDDEN), jnp.float32),
            pltpu.VMEM((CHUNK, HIDDEN), jnp.float32),
            pltpu.VMEM((CHUNK, HIDDEN), jnp.float32),
            pltpu.VMEM((HIDDEN,), jnp.float32),
            pltpu.VMEM((HIDDEN,), jnp.float32),
            pltpu.SMEM((CHUNK,), jnp.float32),
            pltpu.SMEM((CHUNK,), jnp.float32),
            pltpu.SemaphoreType.DMA,
            pltpu.SemaphoreType.DMA,
            pltpu.SemaphoreType.DMA,
            pltpu.SemaphoreType.DMA,
            pltpu.SemaphoreType.DMA,
            pltpu.SemaphoreType.DMA,
            pltpu.SemaphoreType.DMA,
            pltpu.SemaphoreType.DMA,
        ],
    )
    out = k(ids_flat, word_embeddings, ln_gamma, ln_beta)
    return out.reshape(input_ids.shape + (HIDDEN,))
